# baseline (speedup 1.0000x reference)
import functools

import jax
import jax.numpy as jnp
from jax import lax
from jax.experimental import pallas as pl
from jax.experimental.pallas import tpu as pltpu
from jax.experimental.pallas import tpu_sc as plsc

HID = 7168
NC = 2
NS = 16
NW = NC * NS
CHUNK = 4
NBUF = 4


def _emb_body(nchunk, idx_hbm, table_hbm, out_hbm, idx_v, *rest):
    bufs = rest[:NBUF]
    in_sems = rest[NBUF:2 * NBUF]
    out_sems = rest[2 * NBUF:3 * NBUF]
    wid = lax.axis_index("s") * NC + lax.axis_index("c")
    rows_per_w = nchunk * CHUNK
    base = wid * rows_per_w
    pltpu.sync_copy(idx_hbm.at[wid], idx_v)

    in_copies = [None] * NBUF
    out_copies = [None] * NBUF

    for i in range(min(NBUF, nchunk)):
        in_copies[i] = pltpu.async_copy(table_hbm.at[idx_v.at[i]], bufs[i],
                                        in_sems[i])
    for i in range(nchunk):
        b = i % NBUF
        in_copies[b].wait()
        out_copies[b] = pltpu.async_copy(
            bufs[b], out_hbm.at[pl.ds(base + i * CHUNK, CHUNK)], out_sems[b])
        nxt = i + NBUF
        if nxt < nchunk:
            out_copies[b].wait()
            in_copies[b] = pltpu.async_copy(table_hbm.at[idx_v.at[nxt]],
                                            bufs[b], in_sems[b])
    for i in range(max(0, nchunk - NBUF), nchunk):
        out_copies[i % NBUF].wait()


@functools.partial(jax.jit, static_argnames=("ntok",))
def _emb_call(flat_idx, table, ntok):
    nchunk = ntok // (NW * CHUNK)
    mesh = plsc.VectorSubcoreMesh(core_axis_name="c", subcore_axis_name="s")
    k = functools.partial(
        pl.kernel,
        mesh=mesh,
        out_type=jax.ShapeDtypeStruct((ntok, HID), jnp.float32),
        scratch_types=(
            [pltpu.VMEM((nchunk, CHUNK), jnp.int32)]
            + [pltpu.VMEM((CHUNK, HID), jnp.float32)] * NBUF
            + [pltpu.SemaphoreType.DMA] * (2 * NBUF)
        ),
    )(functools.partial(_emb_body, nchunk))
    return k(flat_idx.reshape(NW, nchunk, CHUNK), table)


def kernel(input_ids, embed_tokens):
    ntok = input_ids.size
    flat = input_ids.reshape(-1)
    out = _emb_call(flat, embed_tokens, ntok)
    return out.reshape(1, 1, ntok, embed_tokens.shape[1])


# speedup vs baseline: 1.0018x; 1.0018x over previous
import functools

import jax
import jax.numpy as jnp
from jax import lax
from jax.experimental import pallas as pl
from jax.experimental.pallas import tpu as pltpu
from jax.experimental.pallas import tpu_sc as plsc

HID = 7168
NC = 2
NS = 16
NW = NC * NS
CHUNK = 8
NBUF = 2


def _emb_body(nchunk, idx_hbm, table_hbm, out_hbm, idx_v, *rest):
    bufs = rest[:NBUF]
    in_sems = rest[NBUF:2 * NBUF]
    out_sems = rest[2 * NBUF:3 * NBUF]
    wid = lax.axis_index("s") * NC + lax.axis_index("c")
    rows_per_w = nchunk * CHUNK
    base = wid * rows_per_w
    pltpu.sync_copy(idx_hbm.at[0, 0, 0, pl.ds(base, rows_per_w)], idx_v)

    in_copies = [None] * NBUF
    out_copies = [None] * NBUF

    def gather(i, b):
        return pltpu.async_copy(
            table_hbm.at[idx_v.at[pl.ds(i * CHUNK, CHUNK)]], bufs[b],
            in_sems[b])

    for i in range(min(NBUF, nchunk)):
        in_copies[i] = gather(i, i)
    for i in range(nchunk):
        b = i % NBUF
        in_copies[b].wait()
        out_copies[b] = pltpu.async_copy(
            bufs[b], out_hbm.at[0, 0, pl.ds(base + i * CHUNK, CHUNK)],
            out_sems[b])
        nxt = i + NBUF
        if nxt < nchunk:
            out_copies[b].wait()
            in_copies[b] = gather(nxt, b)
    for i in range(max(0, nchunk - NBUF), nchunk):
        out_copies[i % NBUF].wait()


@functools.partial(jax.jit, static_argnames=("ntok",))
def _emb_call(input_ids, table, ntok):
    nchunk = ntok // (NW * CHUNK)
    mesh = plsc.VectorSubcoreMesh(core_axis_name="c", subcore_axis_name="s")
    k = functools.partial(
        pl.kernel,
        mesh=mesh,
        out_type=jax.ShapeDtypeStruct((1, 1, ntok, HID), jnp.float32),
        scratch_types=(
            [pltpu.VMEM((nchunk * CHUNK,), jnp.int32)]
            + [pltpu.VMEM((CHUNK, HID), jnp.float32)] * NBUF
            + [pltpu.SemaphoreType.DMA] * (2 * NBUF)
        ),
    )(functools.partial(_emb_body, nchunk))
    return k(input_ids, table)


def kernel(input_ids, embed_tokens):
    ntok = input_ids.size
    return _emb_call(input_ids, embed_tokens, ntok)


# speedup vs baseline: 1.0050x; 1.0032x over previous
import functools

import jax
import jax.numpy as jnp
from jax import lax
from jax.experimental import pallas as pl
from jax.experimental.pallas import tpu as pltpu
from jax.experimental.pallas import tpu_sc as plsc

HID = 7168
NC = 2
NS = 16
NW = NC * NS
CHUNK = 8
NBUF = 2


def _emb_body(nchunk, idx_hbm, table_hbm, out_hbm, idx_v, *rest):
    bufs = rest[:NBUF]
    in_sems = rest[NBUF:2 * NBUF]
    out_sems = rest[2 * NBUF:3 * NBUF]
    wid = lax.axis_index("s") * NC + lax.axis_index("c")
    rows_per_w = nchunk * CHUNK
    base = wid * rows_per_w
    pltpu.sync_copy(idx_hbm.at[0, 0, 0, pl.ds(base, rows_per_w)], idx_v)

    in_copies = [None] * NBUF
    out_copies = [None] * NBUF

    def gather(i, b):
        return pltpu.async_copy(
            table_hbm.at[idx_v.at[pl.ds(i * CHUNK, CHUNK)]], bufs[b],
            in_sems[b])

    for i in range(min(NBUF, nchunk)):
        in_copies[i] = gather(i, i)
    for i in range(nchunk):
        b = i % NBUF
        in_copies[b].wait()
        out_copies[b] = pltpu.async_copy(
            bufs[b], out_hbm.at[0, 0, pl.ds(base + i * CHUNK, CHUNK)],
            out_sems[b])
        nxt = i + NBUF
        if nxt < nchunk:
            out_copies[b].wait()
            in_copies[b] = gather(nxt, b)
    for i in range(max(0, nchunk - NBUF), nchunk):
        out_copies[i % NBUF].wait()


@functools.partial(jax.jit, static_argnames=("ntok",))
def _emb_call(input_ids, table, ntok):
    nchunk = ntok // (NW * CHUNK)
    mesh = plsc.VectorSubcoreMesh(core_axis_name="c", subcore_axis_name="s")
    k = functools.partial(
        pl.kernel,
        mesh=mesh,
        out_type=jax.ShapeDtypeStruct((1, 1, ntok, HID), jnp.float32),
        scratch_types=(
            [pltpu.VMEM((nchunk * CHUNK,), jnp.int32)]
            + [pltpu.VMEM((CHUNK, HID), jnp.float32)] * NBUF
            + [pltpu.SemaphoreType.DMA] * (2 * NBUF)
        ),
        compiler_params=pltpu.CompilerParams(use_tc_tiling_on_sc=True),
    )(functools.partial(_emb_body, nchunk))
    return k(input_ids, table)


def kernel(input_ids, embed_tokens):
    ntok = input_ids.size
    return _emb_call(input_ids, embed_tokens, ntok)
